# baseline (speedup 1.0000x reference)
import functools
import math

import jax
import jax.numpy as jnp
from jax.experimental import pallas as pl
from jax.experimental.pallas import tpu as pltpu

_PATCH = 16
_EMBED = 768
_HW = 512
_GRID = _HW // _PATCH
_N = _GRID * _GRID
_WINDOW = 7
_NWIN = 4
_MASK_RATIO = 0.8
_RT = 256
_NG = _N // _RT
_NBUF = 3


def _mask_array():
    H = W_ = _GRID
    all_inds = jnp.arange(H * W_, dtype=jnp.int32).reshape(H, W_)
    pad = _WINDOW // 2
    selectable = all_inds[pad:-pad, pad:-pad].reshape(-1)
    key = jax.random.key(42)
    sampled = jax.random.choice(key, selectable.shape[0], (_NWIN,), replace=False)
    centroids = selectable[sampled]
    off = jnp.arange(int(math.ceil(-_WINDOW / 2)), int(math.ceil(_WINDOW / 2)),
                     dtype=jnp.int32)
    wo = jnp.tile(off[None, :], (_WINDOW, 1))
    sq = jnp.tile((off * H)[None, :], (_WINDOW, 1)).T
    wo = (wo + sq).reshape(1, -1)
    coords = jnp.tile(centroids[:, None], (1, _WINDOW ** 2)) + wo
    n_mask = int(_MASK_RATIO * _WINDOW ** 2)
    inds = coords[:, :n_mask]
    mask = jnp.zeros((_NWIN, _N), jnp.float32)
    mask = mask.at[jnp.arange(_NWIN)[:, None], inds].set(1.0)
    return mask.T.reshape(_NG, _RT, _NWIN)


def _body(xp_ref, w_ref, b_ref, tok_ref, mask_ref, out_hbm, bufs, sems):
    b = pl.program_id(0)
    i = pl.program_id(1)
    step = b * _NG + i
    nsteps = pl.num_programs(0) * pl.num_programs(1)
    slot = step % _NBUF

    @pl.when(step >= _NBUF)
    def _():
        for w in range(_NWIN):
            pltpu.make_async_copy(
                bufs.at[slot, w],
                out_hbm.at[b, w, pl.ds(i * _RT, _RT), :],
                sems.at[slot, w]).wait()

    acc = jnp.dot(xp_ref[0], w_ref[...], preferred_element_type=jnp.float32)
    acc = acc + b_ref[...]
    tok = jnp.broadcast_to(tok_ref[...], acc.shape)
    m = mask_ref[0]
    for w in range(_NWIN):
        sel = jnp.broadcast_to(m[:, w:w + 1] != 0.0, acc.shape)
        bufs[slot, w] = jnp.where(sel, tok, acc)
        pltpu.make_async_copy(
            bufs.at[slot, w],
            out_hbm.at[b, w, pl.ds(i * _RT, _RT), :],
            sems.at[slot, w]).start()

    @pl.when(step == nsteps - 1)
    def _():
        for s in range(_NBUF):
            for w in range(_NWIN):
                pltpu.make_async_copy(
                    bufs.at[s, w],
                    out_hbm.at[b, w, pl.ds(i * _RT, _RT), :],
                    sems.at[s, w]).wait()


def kernel(x, W, b, mask_token):
    Bn = x.shape[0]
    p = _PATCH
    xp = x.reshape(Bn, 3, _GRID, p, _GRID, p)
    xp = xp.transpose(0, 2, 4, 1, 3, 5).reshape(Bn, _N, 3 * p * p)
    Wm = W.reshape(_EMBED, 3 * p * p).T
    mask = _mask_array()
    tok = mask_token.reshape(1, _EMBED)
    b2 = b.reshape(1, _EMBED)

    out = pl.pallas_call(
        _body,
        grid=(Bn, _NG),
        in_specs=[
            pl.BlockSpec((1, _RT, 3 * p * p), lambda bi, i: (bi, i, 0)),
            pl.BlockSpec((3 * p * p, _EMBED), lambda bi, i: (0, 0)),
            pl.BlockSpec((1, _EMBED), lambda bi, i: (0, 0)),
            pl.BlockSpec((1, _EMBED), lambda bi, i: (0, 0)),
            pl.BlockSpec((1, _RT, _NWIN), lambda bi, i: (i, 0, 0)),
        ],
        out_specs=pl.BlockSpec(memory_space=pl.ANY),
        out_shape=jax.ShapeDtypeStruct((Bn, _NWIN, _N, _EMBED), jnp.float32),
        scratch_shapes=[
            pltpu.VMEM((_NBUF, _NWIN, _RT, _EMBED), jnp.float32),
            pltpu.SemaphoreType.DMA((_NBUF, _NWIN)),
        ],
        compiler_params=pltpu.CompilerParams(
            dimension_semantics=("arbitrary", "arbitrary")),
    )(xp, Wm, b2, tok, mask)
    return out


# speedup vs baseline: 1.8586x; 1.8586x over previous
import functools
import math

import jax
import jax.numpy as jnp
from jax.experimental import pallas as pl
from jax.experimental.pallas import tpu as pltpu

_PATCH = 16
_EMBED = 768
_HW = 512
_GRID = _HW // _PATCH
_N = _GRID * _GRID
_WINDOW = 7
_NWIN = 4
_MASK_RATIO = 0.8
_RT = 128
_NG = _N // _RT
_GR = _RT // _GRID
_NBUF = 3


def _mask_array():
    H = W_ = _GRID
    all_inds = jnp.arange(H * W_, dtype=jnp.int32).reshape(H, W_)
    pad = _WINDOW // 2
    selectable = all_inds[pad:-pad, pad:-pad].reshape(-1)
    key = jax.random.key(42)
    sampled = jax.random.choice(key, selectable.shape[0], (_NWIN,), replace=False)
    centroids = selectable[sampled]
    off = jnp.arange(int(math.ceil(-_WINDOW / 2)), int(math.ceil(_WINDOW / 2)),
                     dtype=jnp.int32)
    wo = jnp.tile(off[None, :], (_WINDOW, 1))
    sq = jnp.tile((off * H)[None, :], (_WINDOW, 1)).T
    wo = (wo + sq).reshape(1, -1)
    coords = jnp.tile(centroids[:, None], (1, _WINDOW ** 2)) + wo
    n_mask = int(_MASK_RATIO * _WINDOW ** 2)
    inds = coords[:, :n_mask]
    mask = jnp.zeros((_NWIN, _N), jnp.float32)
    mask = mask.at[jnp.arange(_NWIN)[:, None], inds].set(1.0)
    return mask.T.reshape(_NG, _RT, _NWIN)


def _body(x_ref, w_ref, b_ref, tok_ref, mask_ref, out_hbm, bufs, sems):
    b = pl.program_id(0)
    i = pl.program_id(1)
    step = b * _NG + i
    nsteps = pl.num_programs(0) * pl.num_programs(1)
    slot = step % _NBUF

    @pl.when(step >= _NBUF)
    def _():
        for w in range(_NWIN):
            pltpu.make_async_copy(
                bufs.at[slot, w],
                out_hbm.at[b, w, pl.ds(i * _RT, _RT), :],
                sems.at[slot, w]).wait()

    xr = x_ref[0].reshape(3, _GR, _PATCH, _GRID, _PATCH)
    lhs = xr.transpose(1, 3, 0, 2, 4).reshape(_RT, 3 * _PATCH * _PATCH)
    acc = jnp.dot(lhs, w_ref[...], preferred_element_type=jnp.float32)
    acc = acc + b_ref[...]
    tok = jnp.broadcast_to(tok_ref[...], acc.shape)
    m = mask_ref[0]
    for w in range(_NWIN):
        sel = jnp.broadcast_to(m[:, w:w + 1] != 0.0, acc.shape)
        bufs[slot, w] = jnp.where(sel, tok, acc)
        pltpu.make_async_copy(
            bufs.at[slot, w],
            out_hbm.at[b, w, pl.ds(i * _RT, _RT), :],
            sems.at[slot, w]).start()

    @pl.when(step == nsteps - 1)
    def _():
        for s in range(_NBUF):
            for w in range(_NWIN):
                pltpu.make_async_copy(
                    bufs.at[s, w],
                    out_hbm.at[b, w, pl.ds(i * _RT, _RT), :],
                    sems.at[s, w]).wait()


def kernel(x, W, b, mask_token):
    Bn = x.shape[0]
    p = _PATCH
    Wm = W.reshape(_EMBED, 3 * p * p).T
    mask = _mask_array()
    tok = mask_token.reshape(1, _EMBED)
    b2 = b.reshape(1, _EMBED)

    out = pl.pallas_call(
        _body,
        grid=(Bn, _NG),
        in_specs=[
            pl.BlockSpec((1, 3, _GR * p, _HW), lambda bi, i: (bi, 0, i, 0)),
            pl.BlockSpec((3 * p * p, _EMBED), lambda bi, i: (0, 0)),
            pl.BlockSpec((1, _EMBED), lambda bi, i: (0, 0)),
            pl.BlockSpec((1, _EMBED), lambda bi, i: (0, 0)),
            pl.BlockSpec((1, _RT, _NWIN), lambda bi, i: (i, 0, 0)),
        ],
        out_specs=pl.BlockSpec(memory_space=pl.ANY),
        out_shape=jax.ShapeDtypeStruct((Bn, _NWIN, _N, _EMBED), jnp.float32),
        scratch_shapes=[
            pltpu.VMEM((_NBUF, _NWIN, _RT, _EMBED), jnp.float32),
            pltpu.SemaphoreType.DMA((_NBUF, _NWIN)),
        ],
        compiler_params=pltpu.CompilerParams(
            dimension_semantics=("arbitrary", "arbitrary")),
    )(x, Wm, b2, tok, mask)
    return out


# speedup vs baseline: 2.0533x; 1.1048x over previous
import functools
import math

import jax
import jax.numpy as jnp
from jax import lax
from jax.experimental import pallas as pl
from jax.experimental.pallas import tpu as pltpu
from jax.experimental.pallas import tpu_sc as plsc

_PATCH = 16
_EMBED = 768
_HW = 512
_GRID = _HW // _PATCH
_N = _GRID * _GRID
_WINDOW = 7
_NWIN = 4
_MASK_RATIO = 0.8
_RT = 128
_NG = _N // _RT
_NBUF = 3

_NWORK = 32
_KG = 48
_SUB = 128
_JPP = 24
_PH = 4


def _mask_array():
    H = W_ = _GRID
    all_inds = jnp.arange(H * W_, dtype=jnp.int32).reshape(H, W_)
    pad = _WINDOW // 2
    selectable = all_inds[pad:-pad, pad:-pad].reshape(-1)
    key = jax.random.key(42)
    sampled = jax.random.choice(key, selectable.shape[0], (_NWIN,), replace=False)
    centroids = selectable[sampled]
    off = jnp.arange(int(math.ceil(-_WINDOW / 2)), int(math.ceil(_WINDOW / 2)),
                     dtype=jnp.int32)
    wo = jnp.tile(off[None, :], (_WINDOW, 1))
    sq = jnp.tile((off * H)[None, :], (_WINDOW, 1)).T
    wo = (wo + sq).reshape(1, -1)
    coords = jnp.tile(centroids[:, None], (1, _WINDOW ** 2)) + wo
    n_mask = int(_MASK_RATIO * _WINDOW ** 2)
    inds = coords[:, :n_mask]
    mask = jnp.zeros((_NWIN, _N), jnp.float32)
    mask = mask.at[jnp.arange(_NWIN)[:, None], inds].set(1.0)
    return mask.T.reshape(_NG, _RT, _NWIN)


_QP = 4
_PPH = 64
_HPH = 32


def _shuffle_body(x_hbm, out_hbm, in_v, out_v, isem, osem):
    wid = lax.axis_index("s") * 2 + lax.axis_index("c")
    b = wid // _QP
    q = wid % _QP
    for ph in range(_PH):
        r0 = q * 8 + ph * 2
        for c in range(3):
            pltpu.make_async_copy(
                x_hbm.at[pl.ds((b * 3 + c) * _HW + r0 * _PATCH, _HPH), :],
                in_v.at[pl.ds(c * _HPH, _HPH), :],
                isem).start()
        for c in range(3):
            pltpu.make_async_copy(
                x_hbm.at[pl.ds((b * 3 + c) * _HW + r0 * _PATCH, _HPH), :],
                in_v.at[pl.ds(c * _HPH, _HPH), :],
                isem).wait()

        def _one_patch(t, carry):
            rr = t // _GRID
            s = t % _GRID
            for c in range(3):
                for i2 in range(_PATCH):
                    vec = in_v[c * _HPH + rr * _PATCH + i2,
                               pl.ds(s * _PATCH, _PATCH)]
                    out_v[t, pl.ds((c * _PATCH + i2) * _PATCH, _PATCH)] = vec
            return carry
        lax.fori_loop(0, _PPH, _one_patch, 0)

        n0 = q * (_PH * _PPH) + ph * _PPH
        pltpu.make_async_copy(out_v, out_hbm.at[b, pl.ds(n0, _PPH), :],
                              osem).start()
        pltpu.make_async_copy(out_v, out_hbm.at[b, pl.ds(n0, _PPH), :],
                              osem).wait()


def _im2col_sc(x, Bn):
    x2d = x.reshape(Bn * 3 * _HW, _HW)
    mesh = plsc.VectorSubcoreMesh(core_axis_name="c", subcore_axis_name="s")
    xp = pl.kernel(
        _shuffle_body,
        out_type=jax.ShapeDtypeStruct((Bn, _N, _KG * _PATCH), jnp.float32),
        mesh=mesh,
        scratch_types=[
            pltpu.VMEM((3 * _HPH, _HW), jnp.float32),
            pltpu.VMEM((_PPH, _KG * _PATCH), jnp.float32),
            pltpu.SemaphoreType.DMA,
            pltpu.SemaphoreType.DMA,
        ],
    )(x2d)
    return xp


def _body(xp_ref, w_ref, b_ref, tok_ref, mask_ref, out_hbm, bufs, sems):
    b = pl.program_id(0)
    i = pl.program_id(1)
    step = b * _NG + i
    nsteps = pl.num_programs(0) * pl.num_programs(1)
    slot = step % _NBUF

    @pl.when(step >= _NBUF)
    def _():
        for w in range(_NWIN):
            pltpu.make_async_copy(
                bufs.at[slot, w],
                out_hbm.at[b, w, pl.ds(i * _RT, _RT), :],
                sems.at[slot, w]).wait()

    acc = jnp.dot(xp_ref[0], w_ref[...], preferred_element_type=jnp.float32)
    acc = acc + b_ref[...]
    tok = jnp.broadcast_to(tok_ref[...], acc.shape)
    m = mask_ref[0]
    for w in range(_NWIN):
        sel = jnp.broadcast_to(m[:, w:w + 1] != 0.0, acc.shape)
        bufs[slot, w] = jnp.where(sel, tok, acc)
        pltpu.make_async_copy(
            bufs.at[slot, w],
            out_hbm.at[b, w, pl.ds(i * _RT, _RT), :],
            sems.at[slot, w]).start()

    @pl.when(step == nsteps - 1)
    def _():
        for s in range(_NBUF):
            for w in range(_NWIN):
                pltpu.make_async_copy(
                    bufs.at[s, w],
                    out_hbm.at[b, w, pl.ds(i * _RT, _RT), :],
                    sems.at[s, w]).wait()


def kernel(x, W, b, mask_token):
    Bn = x.shape[0]
    p = _PATCH
    xp = _im2col_sc(x, Bn)
    Wm = W.reshape(_EMBED, 3 * p * p).T
    mask = _mask_array()
    tok = mask_token.reshape(1, _EMBED)
    b2 = b.reshape(1, _EMBED)

    out = pl.pallas_call(
        _body,
        grid=(Bn, _NG),
        in_specs=[
            pl.BlockSpec((1, _RT, 3 * p * p), lambda bi, i: (bi, i, 0)),
            pl.BlockSpec((3 * p * p, _EMBED), lambda bi, i: (0, 0)),
            pl.BlockSpec((1, _EMBED), lambda bi, i: (0, 0)),
            pl.BlockSpec((1, _EMBED), lambda bi, i: (0, 0)),
            pl.BlockSpec((1, _RT, _NWIN), lambda bi, i: (i, 0, 0)),
        ],
        out_specs=pl.BlockSpec(memory_space=pl.ANY),
        out_shape=jax.ShapeDtypeStruct((Bn, _NWIN, _N, _EMBED), jnp.float32),
        scratch_shapes=[
            pltpu.VMEM((_NBUF, _NWIN, _RT, _EMBED), jnp.float32),
            pltpu.SemaphoreType.DMA((_NBUF, _NWIN)),
        ],
        compiler_params=pltpu.CompilerParams(
            dimension_semantics=("arbitrary", "arbitrary")),
    )(xp, Wm, b2, tok, mask)
    return out


# speedup vs baseline: 2.3568x; 1.1478x over previous
import functools
import math

import jax
import jax.numpy as jnp
from jax import lax
from jax.experimental import pallas as pl
from jax.experimental.pallas import tpu as pltpu
from jax.experimental.pallas import tpu_sc as plsc

_PATCH = 16
_EMBED = 768
_HW = 512
_GRID = _HW // _PATCH
_N = _GRID * _GRID
_WINDOW = 7
_NWIN = 4
_MASK_RATIO = 0.8
_RT = 128
_NG = _N // _RT
_NBUF = 3

_NWORK = 32
_KG = 48
_SUB = 128
_JPP = 24
_PH = 4


def _mask_array():
    H = W_ = _GRID
    all_inds = jnp.arange(H * W_, dtype=jnp.int32).reshape(H, W_)
    pad = _WINDOW // 2
    selectable = all_inds[pad:-pad, pad:-pad].reshape(-1)
    key = jax.random.key(42)
    sampled = jax.random.choice(key, selectable.shape[0], (_NWIN,), replace=False)
    centroids = selectable[sampled]
    off = jnp.arange(int(math.ceil(-_WINDOW / 2)), int(math.ceil(_WINDOW / 2)),
                     dtype=jnp.int32)
    wo = jnp.tile(off[None, :], (_WINDOW, 1))
    sq = jnp.tile((off * H)[None, :], (_WINDOW, 1)).T
    wo = (wo + sq).reshape(1, -1)
    coords = jnp.tile(centroids[:, None], (1, _WINDOW ** 2)) + wo
    n_mask = int(_MASK_RATIO * _WINDOW ** 2)
    inds = coords[:, :n_mask]
    mask = jnp.zeros((_NWIN, _N), jnp.float32)
    mask = mask.at[jnp.arange(_NWIN)[:, None], inds].set(1.0)
    return mask.T.reshape(_NG, _RT, _NWIN)


_QP = 4
_PPH = 64
_HPH = 32


def _shuffle_body(x_hbm, out_hbm, in_v, out_v, isem, osem):
    wid = lax.axis_index("s") * 2 + lax.axis_index("c")
    b = wid // _QP
    q = wid % _QP

    def _one_phase(ph, carry):
        r0 = q * 8 + ph * 2
        for c in range(3):
            pltpu.make_async_copy(
                x_hbm.at[pl.ds((b * 3 + c) * _HW + r0 * _PATCH, _HPH), :],
                in_v.at[pl.ds(c * _HPH, _HPH), :],
                isem).start()
        for c in range(3):
            pltpu.make_async_copy(
                x_hbm.at[pl.ds((b * 3 + c) * _HW + r0 * _PATCH, _HPH), :],
                in_v.at[pl.ds(c * _HPH, _HPH), :],
                isem).wait()

        for t in range(_PPH):
            rr = t // _GRID
            s = t % _GRID
            for c in range(3):
                for i2 in range(_PATCH):
                    vec = in_v[c * _HPH + rr * _PATCH + i2,
                               pl.ds(s * _PATCH, _PATCH)]
                    out_v[t, pl.ds((c * _PATCH + i2) * _PATCH, _PATCH)] = vec

        n0 = q * (_PH * _PPH) + ph * _PPH
        pltpu.make_async_copy(out_v, out_hbm.at[b, pl.ds(n0, _PPH), :],
                              osem).start()
        pltpu.make_async_copy(out_v, out_hbm.at[b, pl.ds(n0, _PPH), :],
                              osem).wait()
        return carry

    lax.fori_loop(0, _PH, _one_phase, 0)


def _im2col_sc(x, Bn):
    x2d = x.reshape(Bn * 3 * _HW, _HW)
    mesh = plsc.VectorSubcoreMesh(core_axis_name="c", subcore_axis_name="s")
    xp = pl.kernel(
        _shuffle_body,
        out_type=jax.ShapeDtypeStruct((Bn, _N, _KG * _PATCH), jnp.float32),
        mesh=mesh,
        scratch_types=[
            pltpu.VMEM((3 * _HPH, _HW), jnp.float32),
            pltpu.VMEM((_PPH, _KG * _PATCH), jnp.float32),
            pltpu.SemaphoreType.DMA,
            pltpu.SemaphoreType.DMA,
        ],
    )(x2d)
    return xp


def _body(xp_ref, w_ref, b_ref, tok_ref, mask_ref, out_hbm, bufs, sems):
    b = pl.program_id(0)
    i = pl.program_id(1)
    step = b * _NG + i
    nsteps = pl.num_programs(0) * pl.num_programs(1)
    slot = step % _NBUF

    @pl.when(step >= _NBUF)
    def _():
        for w in range(_NWIN):
            pltpu.make_async_copy(
                bufs.at[slot, w],
                out_hbm.at[b, w, pl.ds(i * _RT, _RT), :],
                sems.at[slot, w]).wait()

    acc = jnp.dot(xp_ref[0], w_ref[...], preferred_element_type=jnp.float32)
    acc = acc + b_ref[...]
    tok = jnp.broadcast_to(tok_ref[...], acc.shape)
    m = mask_ref[0]
    for w in range(_NWIN):
        sel = jnp.broadcast_to(m[:, w:w + 1] != 0.0, acc.shape)
        bufs[slot, w] = jnp.where(sel, tok, acc)
        pltpu.make_async_copy(
            bufs.at[slot, w],
            out_hbm.at[b, w, pl.ds(i * _RT, _RT), :],
            sems.at[slot, w]).start()

    @pl.when(step == nsteps - 1)
    def _():
        for s in range(_NBUF):
            for w in range(_NWIN):
                pltpu.make_async_copy(
                    bufs.at[s, w],
                    out_hbm.at[b, w, pl.ds(i * _RT, _RT), :],
                    sems.at[s, w]).wait()


def kernel(x, W, b, mask_token):
    Bn = x.shape[0]
    p = _PATCH
    xp = _im2col_sc(x, Bn)
    Wm = W.reshape(_EMBED, 3 * p * p).T
    mask = _mask_array()
    tok = mask_token.reshape(1, _EMBED)
    b2 = b.reshape(1, _EMBED)

    out = pl.pallas_call(
        _body,
        grid=(Bn, _NG),
        in_specs=[
            pl.BlockSpec((1, _RT, 3 * p * p), lambda bi, i: (bi, i, 0)),
            pl.BlockSpec((3 * p * p, _EMBED), lambda bi, i: (0, 0)),
            pl.BlockSpec((1, _EMBED), lambda bi, i: (0, 0)),
            pl.BlockSpec((1, _EMBED), lambda bi, i: (0, 0)),
            pl.BlockSpec((1, _RT, _NWIN), lambda bi, i: (i, 0, 0)),
        ],
        out_specs=pl.BlockSpec(memory_space=pl.ANY),
        out_shape=jax.ShapeDtypeStruct((Bn, _NWIN, _N, _EMBED), jnp.float32),
        scratch_shapes=[
            pltpu.VMEM((_NBUF, _NWIN, _RT, _EMBED), jnp.float32),
            pltpu.SemaphoreType.DMA((_NBUF, _NWIN)),
        ],
        compiler_params=pltpu.CompilerParams(
            dimension_semantics=("arbitrary", "arbitrary")),
    )(xp, Wm, b2, tok, mask)
    return out


# speedup vs baseline: 3.0537x; 1.2957x over previous
import functools
import math

import jax
import jax.numpy as jnp
from jax import lax
from jax.experimental import pallas as pl
from jax.experimental.pallas import tpu as pltpu
from jax.experimental.pallas import tpu_sc as plsc

_PATCH = 16
_EMBED = 768
_HW = 512
_GRID = _HW // _PATCH
_N = _GRID * _GRID
_WINDOW = 7
_NWIN = 4
_MASK_RATIO = 0.8
_RT = 256
_NG = _N // _RT
_NBUF = 3

_NWORK = 32
_KG = 48
_SUB = 128
_JPP = 24
_PH = 4


def _mask_array():
    H = W_ = _GRID
    all_inds = jnp.arange(H * W_, dtype=jnp.int32).reshape(H, W_)
    pad = _WINDOW // 2
    selectable = all_inds[pad:-pad, pad:-pad].reshape(-1)
    key = jax.random.key(42)
    sampled = jax.random.choice(key, selectable.shape[0], (_NWIN,), replace=False)
    centroids = selectable[sampled]
    off = jnp.arange(int(math.ceil(-_WINDOW / 2)), int(math.ceil(_WINDOW / 2)),
                     dtype=jnp.int32)
    wo = jnp.tile(off[None, :], (_WINDOW, 1))
    sq = jnp.tile((off * H)[None, :], (_WINDOW, 1)).T
    wo = (wo + sq).reshape(1, -1)
    coords = jnp.tile(centroids[:, None], (1, _WINDOW ** 2)) + wo
    n_mask = int(_MASK_RATIO * _WINDOW ** 2)
    inds = coords[:, :n_mask]
    mask = jnp.zeros((_NWIN, _N), jnp.float32)
    mask = mask.at[jnp.arange(_NWIN)[:, None], inds].set(1.0)
    return mask.T.reshape(_NG, _RT, _NWIN)


_QP = 4
_PPH = 32
_SPH = 8
_DP = _SPH // 2


def _in_copy(x_hbm, in_v, isem, b, r, slot, c):
    return pltpu.make_async_copy(
        x_hbm.at[pl.ds((b * 3 + c) * _HW + r * _PATCH, _PATCH), :],
        in_v.at[slot, pl.ds(c * _PATCH, _PATCH), :],
        isem.at[slot])


def _out_copy(out_hbm, out_v, osem, b, n0, slot):
    return pltpu.make_async_copy(
        out_v.at[slot], out_hbm.at[b, pl.ds(n0, _PPH), :], osem.at[slot])


def _shuffle_body(x_hbm, out_hbm, in_v, out_v, isem, osem):
    wid = lax.axis_index("s") * 2 + lax.axis_index("c")
    b = wid // _QP
    q = wid % _QP
    r_base = q * _SPH
    n_base = q * (_SPH * _PPH)

    for slot in range(2):
        for c in range(3):
            _in_copy(x_hbm, in_v, isem, b, r_base + slot, slot, c).start()

    def _double_phase(dp, carry):
        for slot in range(2):
            ph = dp * 2 + slot
            r = r_base + ph
            n0 = n_base + ph * _PPH
            for c in range(3):
                _in_copy(x_hbm, in_v, isem, b, r, slot, c).wait()

            @pl.when(dp >= 1)
            def _():
                _out_copy(out_hbm, out_v, osem, b, n0 - 2 * _PPH, slot).wait()

            for s in range(_PPH):
                for c in range(3):
                    for i2 in range(_PATCH):
                        vec = in_v[slot, c * _PATCH + i2,
                                   pl.ds(s * _PATCH, _PATCH)]
                        out_v[slot, s,
                              pl.ds((c * _PATCH + i2) * _PATCH, _PATCH)] = vec

            _out_copy(out_hbm, out_v, osem, b, n0, slot).start()

            @pl.when(dp < _DP - 1)
            def _():
                for c in range(3):
                    _in_copy(x_hbm, in_v, isem, b, r + 2, slot, c).start()
        return carry

    lax.fori_loop(0, _DP, _double_phase, 0)
    for slot in range(2):
        ph = _SPH - 2 + slot
        _out_copy(out_hbm, out_v, osem, b, n_base + ph * _PPH, slot).wait()


def _im2col_sc(x, Bn):
    x2d = x.reshape(Bn * 3 * _HW, _HW)
    mesh = plsc.VectorSubcoreMesh(core_axis_name="c", subcore_axis_name="s")
    xp = pl.kernel(
        _shuffle_body,
        out_type=jax.ShapeDtypeStruct((Bn, _N, _KG * _PATCH), jnp.float32),
        mesh=mesh,
        scratch_types=[
            pltpu.VMEM((2, 3 * _PATCH, _HW), jnp.float32),
            pltpu.VMEM((2, _PPH, _KG * _PATCH), jnp.float32),
            pltpu.SemaphoreType.DMA((2,)),
            pltpu.SemaphoreType.DMA((2,)),
        ],
    )(x2d)
    return xp


def _body(xp_ref, w_ref, b_ref, tok_ref, mask_ref, out_hbm, bufs, sems):
    b = pl.program_id(0)
    i = pl.program_id(1)
    step = b * _NG + i
    nsteps = pl.num_programs(0) * pl.num_programs(1)
    slot = step % _NBUF

    @pl.when(step >= _NBUF)
    def _():
        for w in range(_NWIN):
            pltpu.make_async_copy(
                bufs.at[slot, w],
                out_hbm.at[b, w, pl.ds(i * _RT, _RT), :],
                sems.at[slot, w]).wait()

    acc = jnp.dot(xp_ref[0], w_ref[...], preferred_element_type=jnp.float32)
    acc = acc + b_ref[...]
    tok = jnp.broadcast_to(tok_ref[...], acc.shape)
    m = mask_ref[0]
    for w in range(_NWIN):
        sel = jnp.broadcast_to(m[:, w:w + 1] != 0.0, acc.shape)
        bufs[slot, w] = jnp.where(sel, tok, acc)
        pltpu.make_async_copy(
            bufs.at[slot, w],
            out_hbm.at[b, w, pl.ds(i * _RT, _RT), :],
            sems.at[slot, w]).start()

    @pl.when(step == nsteps - 1)
    def _():
        for s in range(_NBUF):
            for w in range(_NWIN):
                pltpu.make_async_copy(
                    bufs.at[s, w],
                    out_hbm.at[b, w, pl.ds(i * _RT, _RT), :],
                    sems.at[s, w]).wait()


def kernel(x, W, b, mask_token):
    Bn = x.shape[0]
    p = _PATCH
    xp = _im2col_sc(x, Bn)
    Wm = W.reshape(_EMBED, 3 * p * p).T
    mask = _mask_array()
    tok = mask_token.reshape(1, _EMBED)
    b2 = b.reshape(1, _EMBED)

    out = pl.pallas_call(
        _body,
        grid=(Bn, _NG),
        in_specs=[
            pl.BlockSpec((1, _RT, 3 * p * p), lambda bi, i: (bi, i, 0)),
            pl.BlockSpec((3 * p * p, _EMBED), lambda bi, i: (0, 0)),
            pl.BlockSpec((1, _EMBED), lambda bi, i: (0, 0)),
            pl.BlockSpec((1, _EMBED), lambda bi, i: (0, 0)),
            pl.BlockSpec((1, _RT, _NWIN), lambda bi, i: (i, 0, 0)),
        ],
        out_specs=pl.BlockSpec(memory_space=pl.ANY),
        out_shape=jax.ShapeDtypeStruct((Bn, _NWIN, _N, _EMBED), jnp.float32),
        scratch_shapes=[
            pltpu.VMEM((_NBUF, _NWIN, _RT, _EMBED), jnp.float32),
            pltpu.SemaphoreType.DMA((_NBUF, _NWIN)),
        ],
        compiler_params=pltpu.CompilerParams(
            dimension_semantics=("arbitrary", "arbitrary")),
    )(xp, Wm, b2, tok, mask)
    return out
